# baseline (speedup 1.0000x reference)
import functools

import jax
import jax.numpy as jnp
from jax import lax
from jax.experimental import pallas as pl
from jax.experimental.pallas import tpu as pltpu
from jax.experimental.pallas import tpu_sc as plsc

D = 128
CHUNK = 64
NBUF = 5
AHEAD = 2
SEGS = 4
TC_BLOCK = 1024


def _sc_gather_body(idx_hbm, table_hbm, g_hbm, idx_all, g_v, sem_g, sem_out,
                    *, rows_per_worker, num_cores):
    wid = lax.axis_index("s") * num_cores + lax.axis_index("c")
    base = wid * rows_per_worker
    n_chunks = rows_per_worker // CHUNK

    pltpu.sync_copy(idx_hbm.at[pl.ds(base, rows_per_worker)], idx_all)

    def fire_in(s, k):
        pltpu.async_copy(table_hbm.at[idx_all.at[pl.ds(k * CHUNK, CHUNK)]],
                         g_v.at[s], sem_g.at[s])

    def wait_in(s, k):
        pltpu.make_async_copy(table_hbm.at[idx_all.at[pl.ds(k * CHUNK, CHUNK)]],
                              g_v.at[s], sem_g.at[s]).wait()

    def fire_out(s, k):
        pltpu.async_copy(g_v.at[s], g_hbm.at[pl.ds(base + k * CHUNK, CHUNK)],
                         sem_out.at[s])

    def wait_out(s, k):
        pltpu.make_async_copy(g_v.at[s],
                              g_hbm.at[pl.ds(base + k * CHUNK, CHUNK)],
                              sem_out.at[s]).wait()

    for k in range(AHEAD):
        fire_in(k % NBUF, k)

    def outer(k0, carry):
        for s in range(NBUF):
            k = k0 * NBUF + s
            t = (s + AHEAD) % NBUF

            @pl.when(k + AHEAD < n_chunks)
            def _():
                @pl.when(k + AHEAD >= NBUF)
                def _():
                    wait_out(t, k + AHEAD - NBUF)
                fire_in(t, k + AHEAD)

            wait_in(s, k)
            fire_out(s, k)
        return carry

    lax.fori_loop(0, n_chunks // NBUF, outer, 0)
    for s in range(NBUF):
        wait_out(s, n_chunks - NBUF + s)


def _make_sc_gather(seg_rows):
    info = plsc.get_sparse_core_info()
    nw = info.num_cores * info.num_subcores
    rows_per_worker = seg_rows // nw
    assert rows_per_worker % (CHUNK * NBUF) == 0

    mesh = plsc.VectorSubcoreMesh(core_axis_name="c", subcore_axis_name="s")
    return pl.kernel(
        functools.partial(_sc_gather_body, rows_per_worker=rows_per_worker,
                          num_cores=info.num_cores),
        out_type=jax.ShapeDtypeStruct((seg_rows, D), jnp.float32),
        mesh=mesh,
        scratch_types=[
            pltpu.VMEM((rows_per_worker,), jnp.int32),
            pltpu.VMEM((NBUF, CHUNK, D), jnp.float32),
            pltpu.SemaphoreType.DMA((NBUF,)),
            pltpu.SemaphoreType.DMA((NBUF,)),
        ],
    )


def _tc_add_first_body(x_ref, p_ref, g_ref, o_ref):
    o_ref[...] = x_ref[...] + p_ref[...] + g_ref[...]


def _tc_add_rest_body(prev_ref, x_ref, p_ref, g_ref, o_ref):
    del prev_ref
    o_ref[...] = x_ref[...] + p_ref[...] + g_ref[...]


def _make_tc_add(n, seg_rows, seg_base, first):
    grid = (seg_rows // TC_BLOCK,)
    blk = (TC_BLOCK, D)
    seg0 = seg_base // TC_BLOCK
    xp_spec = pl.BlockSpec(blk, lambda j: (seg0 + j, 0))
    g_spec = pl.BlockSpec(blk, lambda j: (j, 0))
    out_spec = pl.BlockSpec(blk, lambda j: (seg0 + j, 0))
    if first:
        return pl.pallas_call(
            _tc_add_first_body,
            grid=grid,
            in_specs=[xp_spec, xp_spec, g_spec],
            out_specs=out_spec,
            out_shape=jax.ShapeDtypeStruct((n, D), jnp.float32),
        )
    prev_spec = pl.BlockSpec((8, D), lambda j: (0, 0))
    return pl.pallas_call(
        _tc_add_rest_body,
        grid=grid,
        in_specs=[prev_spec, xp_spec, xp_spec, g_spec],
        out_specs=out_spec,
        out_shape=jax.ShapeDtypeStruct((n, D), jnp.float32),
        input_output_aliases={0: 0},
    )


def kernel(x, variable, pos_emb, variable_table):
    B, S, d = x.shape
    n = B * S
    xf = x.reshape(n, d)
    pf = pos_emb.reshape(n, d)
    idx = variable.reshape(n).astype(jnp.int32)

    seg_rows = n // SEGS
    sc_gather = _make_sc_gather(seg_rows)

    gs = [sc_gather(idx[i * seg_rows:(i + 1) * seg_rows], variable_table)
          for i in range(SEGS)]

    out = _make_tc_add(n, seg_rows, 0, True)(xf, pf, gs[0])
    for i in range(1, SEGS):
        out = _make_tc_add(n, seg_rows, i * seg_rows, False)(
            out, xf, pf, gs[i])
    return out.reshape(B, S, d)


# speedup vs baseline: 1.6494x; 1.6494x over previous
import functools

import jax
import jax.numpy as jnp
from jax import lax
from jax.experimental import pallas as pl
from jax.experimental.pallas import tpu as pltpu
from jax.experimental.pallas import tpu_sc as plsc

D = 128
CHUNK = 64
NBUF = 4
AHEAD = 2


def _body(x_hbm, idx_hbm, pos_hbm, table_hbm, out_hbm,
          idx_all, g_v, x_v, p_v, sem_g, sem_xp, sem_out,
          *, rows_per_worker, num_cores):
    wid = lax.axis_index("s") * num_cores + lax.axis_index("c")
    base = wid * rows_per_worker
    n_chunks = rows_per_worker // CHUNK

    pltpu.sync_copy(idx_hbm.at[pl.ds(base, rows_per_worker)], idx_all)

    def fire_in(s, k):
        row0 = base + k * CHUNK
        pltpu.async_copy(table_hbm.at[idx_all.at[pl.ds(k * CHUNK, CHUNK)]],
                         g_v.at[s], sem_g.at[s])
        pltpu.async_copy(x_hbm.at[pl.ds(row0, CHUNK)], x_v.at[s], sem_xp.at[s])
        pltpu.async_copy(pos_hbm.at[pl.ds(row0, CHUNK)], p_v.at[s], sem_xp.at[s])

    def wait_in(s, k):
        pltpu.make_async_copy(table_hbm.at[idx_all.at[pl.ds(k * CHUNK, CHUNK)]],
                              g_v.at[s], sem_g.at[s]).wait()
        row0 = base + k * CHUNK
        pltpu.make_async_copy(x_hbm.at[pl.ds(row0, CHUNK)], x_v.at[s],
                              sem_xp.at[s]).wait()
        pltpu.make_async_copy(pos_hbm.at[pl.ds(row0, CHUNK)], p_v.at[s],
                              sem_xp.at[s]).wait()

    def fire_out(s, k):
        row0 = base + k * CHUNK
        pltpu.async_copy(g_v.at[s], out_hbm.at[pl.ds(row0, CHUNK)],
                         sem_out.at[s])

    def wait_out(s, k):
        row0 = base + k * CHUNK
        pltpu.make_async_copy(g_v.at[s], out_hbm.at[pl.ds(row0, CHUNK)],
                              sem_out.at[s]).wait()

    def compute(s):
        def vec_body(i, carry):
            r = i >> 3
            c = (i & 7) * 16
            sl = pl.ds(c, 16)
            plsc.addupdate(g_v.at[s, r, sl], x_v[s, r, sl] + p_v[s, r, sl])
            return carry

        lax.fori_loop(0, CHUNK * (D // 16), vec_body, 0, unroll=8)

    for k in range(AHEAD):
        fire_in(k % NBUF, k)

    def outer(k0, carry):
        for s in range(NBUF):
            k = k0 * NBUF + s
            t = (s + AHEAD) % NBUF

            @pl.when(k + AHEAD < n_chunks)
            def _():
                @pl.when(k + AHEAD >= NBUF)
                def _():
                    wait_out(t, k + AHEAD - NBUF)
                fire_in(t, k + AHEAD)

            wait_in(s, k)
            compute(s)
            fire_out(s, k)
        return carry

    lax.fori_loop(0, n_chunks // NBUF, outer, 0)

    for s in range(NBUF):
        wait_out(s, n_chunks - NBUF + s)


def kernel(x, variable, pos_emb, variable_table):
    B, S, d = x.shape
    n = B * S
    xf = x.reshape(n, d)
    pf = pos_emb.reshape(n, d)
    idx = variable.reshape(n).astype(jnp.int32)

    info = plsc.get_sparse_core_info()
    nw = info.num_cores * info.num_subcores
    rows_per_worker = n // nw
    assert rows_per_worker % (CHUNK * NBUF) == 0

    mesh = plsc.VectorSubcoreMesh(core_axis_name="c", subcore_axis_name="s")
    run = pl.kernel(
        functools.partial(_body, rows_per_worker=rows_per_worker,
                          num_cores=info.num_cores),
        out_type=jax.ShapeDtypeStruct((n, d), jnp.float32),
        mesh=mesh,
        scratch_types=[
            pltpu.VMEM((rows_per_worker,), jnp.int32),
            pltpu.VMEM((NBUF, CHUNK, D), jnp.float32),
            pltpu.VMEM((NBUF, CHUNK, D), jnp.float32),
            pltpu.VMEM((NBUF, CHUNK, D), jnp.float32),
            pltpu.SemaphoreType.DMA((NBUF,)),
            pltpu.SemaphoreType.DMA((NBUF,)),
            pltpu.SemaphoreType.DMA((NBUF,)),
        ],
    )
    out = run(xf, idx, pf, variable_table)
    return out.reshape(B, S, d)


# speedup vs baseline: 1.6594x; 1.0061x over previous
import functools

import jax
import jax.numpy as jnp
from jax import lax
from jax.experimental import pallas as pl
from jax.experimental.pallas import tpu as pltpu
from jax.experimental.pallas import tpu_sc as plsc

D = 128
CHUNK = 32
NBUF = 8
AHEAD = 4


def _body(x_hbm, idx_hbm, pos_hbm, table_hbm, out_hbm,
          idx_all, g_v, x_v, p_v, sem_g, sem_xp, sem_out,
          *, rows_per_worker, num_cores):
    wid = lax.axis_index("s") * num_cores + lax.axis_index("c")
    base = wid * rows_per_worker
    n_chunks = rows_per_worker // CHUNK

    pltpu.sync_copy(idx_hbm.at[pl.ds(base, rows_per_worker)], idx_all)

    def fire_in(s, k):
        row0 = base + k * CHUNK
        pltpu.async_copy(table_hbm.at[idx_all.at[pl.ds(k * CHUNK, CHUNK)]],
                         g_v.at[s], sem_g.at[s])
        pltpu.async_copy(x_hbm.at[pl.ds(row0, CHUNK)], x_v.at[s], sem_xp.at[s])
        pltpu.async_copy(pos_hbm.at[pl.ds(row0, CHUNK)], p_v.at[s], sem_xp.at[s])

    def wait_in(s, k):
        pltpu.make_async_copy(table_hbm.at[idx_all.at[pl.ds(k * CHUNK, CHUNK)]],
                              g_v.at[s], sem_g.at[s]).wait()
        row0 = base + k * CHUNK
        pltpu.make_async_copy(x_hbm.at[pl.ds(row0, CHUNK)], x_v.at[s],
                              sem_xp.at[s]).wait()
        pltpu.make_async_copy(pos_hbm.at[pl.ds(row0, CHUNK)], p_v.at[s],
                              sem_xp.at[s]).wait()

    def fire_out(s, k):
        row0 = base + k * CHUNK
        pltpu.async_copy(g_v.at[s], out_hbm.at[pl.ds(row0, CHUNK)],
                         sem_out.at[s])

    def wait_out(s, k):
        row0 = base + k * CHUNK
        pltpu.make_async_copy(g_v.at[s], out_hbm.at[pl.ds(row0, CHUNK)],
                              sem_out.at[s]).wait()

    def compute(s):
        def vec_body(i, carry):
            r = i >> 3
            c = (i & 7) * 16
            sl = pl.ds(c, 16)
            plsc.addupdate(g_v.at[s, r, sl], x_v[s, r, sl] + p_v[s, r, sl])
            return carry

        lax.fori_loop(0, CHUNK * (D // 16), vec_body, 0, unroll=8)

    for k in range(AHEAD):
        fire_in(k % NBUF, k)

    def outer(k0, carry):
        for s in range(NBUF):
            k = k0 * NBUF + s
            t = (s + AHEAD) % NBUF

            @pl.when(k + AHEAD < n_chunks)
            def _():
                @pl.when(k + AHEAD >= NBUF)
                def _():
                    wait_out(t, k + AHEAD - NBUF)
                fire_in(t, k + AHEAD)

            wait_in(s, k)
            compute(s)
            fire_out(s, k)
        return carry

    lax.fori_loop(0, n_chunks // NBUF, outer, 0)

    for s in range(NBUF):
        wait_out(s, n_chunks - NBUF + s)


def kernel(x, variable, pos_emb, variable_table):
    B, S, d = x.shape
    n = B * S
    xf = x.reshape(n, d)
    pf = pos_emb.reshape(n, d)
    idx = variable.reshape(n).astype(jnp.int32)

    info = plsc.get_sparse_core_info()
    nw = info.num_cores * info.num_subcores
    rows_per_worker = n // nw
    assert rows_per_worker % (CHUNK * NBUF) == 0

    mesh = plsc.VectorSubcoreMesh(core_axis_name="c", subcore_axis_name="s")
    run = pl.kernel(
        functools.partial(_body, rows_per_worker=rows_per_worker,
                          num_cores=info.num_cores),
        out_type=jax.ShapeDtypeStruct((n, d), jnp.float32),
        mesh=mesh,
        scratch_types=[
            pltpu.VMEM((rows_per_worker,), jnp.int32),
            pltpu.VMEM((NBUF, CHUNK, D), jnp.float32),
            pltpu.VMEM((NBUF, CHUNK, D), jnp.float32),
            pltpu.VMEM((NBUF, CHUNK, D), jnp.float32),
            pltpu.SemaphoreType.DMA((NBUF,)),
            pltpu.SemaphoreType.DMA((NBUF,)),
            pltpu.SemaphoreType.DMA((NBUF,)),
        ],
    )
    out = run(xf, idx, pf, variable_table)
    return out.reshape(B, S, d)
